# baseline (speedup 1.0000x reference)
import jax
import jax.numpy as jnp
from jax import lax
from jax.experimental import pallas as pl
from jax.experimental.pallas import tpu as pltpu
from jax.experimental.pallas import tpu_sc as plsc

N_NODES = 50000
IN_CH = 256
NUM_G = 512
LANES = 16
BLK = 80
NBLK = N_NODES // BLK
NWORK = 32
STEPS = -(-NBLK // NWORK)
NJ = IN_CH // LANES
NQ = BLK // LANES

_DNUMS = lax.GatherDimensionNumbers(
    offset_dims=(), collapsed_slice_dims=(0,), start_index_map=(0,))


def _xlane(v, idx):
    return lax.gather(v, idx[:, None], _DNUMS, (1,),
                      mode=lax.GatherScatterMode.PROMISE_IN_BOUNDS)


def _sc_pool_body(x_hbm, batch_hbm, wg_hbm, out_hbm, gout_hbm,
                  wv, idxv, xblk, sblk, estage, gsum, acc,
                  lsem0, lsem1, ssem0, ssem1):
    c = lax.axis_index("c")
    s = lax.axis_index("s")
    w = s * 2 + c
    lsem = (lsem0, lsem1)
    ssem = (ssem0, ssem1)

    pltpu.sync_copy(wg_hbm, wv)
    wr = [wv[pl.ds(LANES * j, LANES)] for j in range(NJ)]
    lane = lax.iota(jnp.int32, LANES)
    perms = [lane ^ m for m in (8, 4, 2, 1)]
    zeroi = jnp.zeros((LANES,), jnp.int32)

    zero = jnp.zeros((LANES,), jnp.float32)

    def zrow(r, carry):
        for j in range(NJ):
            sblk[0, r, pl.ds(LANES * j, LANES)] = zero
        return carry

    lax.fori_loop(0, 32, zrow, 0)
    for q in range(NUM_G // LANES):
        gsum[pl.ds(LANES * q, LANES)] = zero
    pltpu.sync_copy(sblk.at[0, pl.ds(0, 32)], acc.at[pl.ds(s * 32, 32)])
    plsc.subcore_barrier()

    def blk_of(k):
        return k * NWORK + w

    def load_start(k):
        buf, slot, b = k % 2, k % 4, blk_of(k)
        pltpu.async_copy(batch_hbm.at[pl.ds(b * BLK, BLK)], idxv.at[slot],
                         lsem[buf])
        pltpu.async_copy(x_hbm.at[pl.ds(b * BLK, BLK)], xblk.at[buf],
                         lsem[buf])

    def load_wait(k):
        buf, slot, b = k % 2, k % 4, blk_of(k)
        pltpu.make_async_copy(batch_hbm.at[pl.ds(b * BLK, BLK)],
                              idxv.at[slot], lsem[buf]).wait()
        pltpu.make_async_copy(x_hbm.at[pl.ds(b * BLK, BLK)],
                              xblk.at[buf], lsem[buf]).wait()

    def scatter_start(k):
        buf, slot = k % 2, k % 4
        pltpu.async_copy(sblk.at[buf], acc.at[idxv.at[slot]], ssem[buf],
                         add=True)

    def scatter_wait(k):
        buf, slot = k % 2, k % 4
        pltpu.make_async_copy(sblk.at[buf], acc.at[idxv.at[slot]],
                              ssem[buf]).wait()

    def compute(k):
        buf, slot = k % 2, k % 4

        def row(r, carry):
            xr = [xblk[buf, r, pl.ds(LANES * j, LANES)] for j in range(NJ)]
            ps = [xr[j] * wr[j] for j in range(NJ)]
            while len(ps) > 1:
                ps = [ps[i] + ps[i + 1] for i in range(0, len(ps), 2)]
            tot = ps[0]
            for p in perms:
                tot = tot + _xlane(tot, p)
            ev = jnp.exp(tot)
            for j in range(NJ):
                sblk[buf, r, pl.ds(LANES * j, LANES)] = xr[j] * ev
            estage[r, pl.ds(0, LANES)] = ev
            return carry

        lax.fori_loop(0, BLK, row, 0)

        for q in range(NQ):
            eq = plsc.load_gather(estage, [lane + (LANES * q), zeroi])
            idxr = idxv[slot, pl.ds(LANES * q, LANES)]
            plsc.addupdate_scatter(gsum, [idxr], eq)

    conds = [blk_of(k) < NBLK for k in range(STEPS)]

    pl.when(conds[0])(lambda: load_start(0))
    for k in range(STEPS):
        if k + 1 < STEPS:
            pl.when(conds[k + 1])(lambda k=k: load_start(k + 1))
        if k >= 2:
            pl.when(conds[k - 2])(lambda k=k: scatter_wait(k - 2))

        def step(k=k):
            load_wait(k)
            compute(k)
            scatter_start(k)

        pl.when(conds[k])(step)

    for j in (STEPS - 2, STEPS - 1):
        pl.when(conds[j])(lambda j=j: scatter_wait(j))

    plsc.subcore_barrier()
    pltpu.sync_copy(acc.at[pl.ds(s * 32, 32)], out_hbm.at[c, pl.ds(s * 32, 32)])
    pltpu.sync_copy(gsum, gout_hbm.at[c, s])


def _finish_body(p_ref, g_ref, w_ref, b_ref, o_ref):
    a = p_ref[0] + p_ref[1]
    gs = jnp.sum(g_ref[...], axis=(0, 1))
    gsc = jnp.transpose(gs.reshape(1, NUM_G))
    nonempty = gsc > 0.0
    inv = jnp.where(nonempty, 1.0 / jnp.where(nonempty, gsc, 1.0), 0.0)
    pooled = a * inv
    out = jnp.dot(pooled, w_ref[...], preferred_element_type=jnp.float32)
    o_ref[...] = out + jnp.where(nonempty, b_ref[...], 0.0)


def kernel(x, batch, W_gate, b_gate, W_nn, b_nn):
    del b_gate
    batch32 = batch.astype(jnp.int32)
    wg = W_gate.reshape(IN_CH)

    mesh = plsc.VectorSubcoreMesh(core_axis_name="c", subcore_axis_name="s")
    sc_pool = pl.kernel(
        _sc_pool_body,
        mesh=mesh,
        compiler_params=pltpu.CompilerParams(
            needs_layout_passes=False, use_tc_tiling_on_sc=False),
        out_type=(
            jax.ShapeDtypeStruct((2, NUM_G, IN_CH), jnp.float32),
            jax.ShapeDtypeStruct((2, LANES, NUM_G), jnp.float32),
        ),
        scratch_types=[
            pltpu.VMEM((IN_CH,), jnp.float32),
            pltpu.VMEM((4, BLK), jnp.int32),
            pltpu.VMEM((2, BLK, IN_CH), jnp.float32),
            pltpu.VMEM((2, BLK, IN_CH), jnp.float32),
            pltpu.VMEM((BLK, LANES), jnp.float32),
            pltpu.VMEM((NUM_G,), jnp.float32),
            pltpu.VMEM_SHARED((NUM_G, IN_CH), jnp.float32),
            pltpu.SemaphoreType.DMA,
            pltpu.SemaphoreType.DMA,
            pltpu.SemaphoreType.DMA,
            pltpu.SemaphoreType.DMA,
        ],
    )
    partials, gparts = sc_pool(x, batch32, wg)

    out = pl.pallas_call(
        _finish_body,
        out_shape=jax.ShapeDtypeStruct((NUM_G, 2 * IN_CH), jnp.float32),
    )(partials, gparts, W_nn, b_nn.reshape(1, 2 * IN_CH))
    return out


# speedup vs baseline: 1.2214x; 1.2214x over previous
import jax
import jax.numpy as jnp
from jax import lax
from jax.experimental import pallas as pl
from jax.experimental.pallas import tpu as pltpu
from jax.experimental.pallas import tpu_sc as plsc

N_NODES = 50000
IN_CH = 256
NUM_G = 512
LANES = 16
BLK = 80
NBLK = N_NODES // BLK
NWORK = 32
STEPS = -(-NBLK // NWORK)
NJ = IN_CH // LANES
NQ = BLK // LANES

_DNUMS = lax.GatherDimensionNumbers(
    offset_dims=(), collapsed_slice_dims=(0,), start_index_map=(0,))


def _xlane(v, idx):
    return lax.gather(v, idx[:, None], _DNUMS, (1,),
                      mode=lax.GatherScatterMode.PROMISE_IN_BOUNDS)


def _sc_pool_body(x_hbm, batch_hbm, wg_hbm, out_hbm, gout_hbm,
                  wv, idxv, xblk, sblk, estage, gsum, acc,
                  lsem0, lsem1, ssem0, ssem1):
    c = lax.axis_index("c")
    s = lax.axis_index("s")
    w = s * 2 + c
    lsem = (lsem0, lsem1)
    ssem = (ssem0, ssem1)

    pltpu.sync_copy(wg_hbm, wv)
    wr = [wv[pl.ds(LANES * j, LANES)] for j in range(NJ)]
    lane = lax.iota(jnp.int32, LANES)
    perms = [lane ^ m for m in (8, 4, 2, 1)]
    zeroi = jnp.zeros((LANES,), jnp.int32)

    zero = jnp.zeros((LANES,), jnp.float32)

    def zrow(r, carry):
        for j in range(NJ):
            sblk[0, r, pl.ds(LANES * j, LANES)] = zero
        return carry

    lax.fori_loop(0, 32, zrow, 0)
    for q in range(NUM_G // LANES):
        gsum[pl.ds(LANES * q, LANES)] = zero
    pltpu.sync_copy(sblk.at[0, pl.ds(0, 32)], acc.at[pl.ds(s * 32, 32)])
    plsc.subcore_barrier()

    def blk_of(k):
        return k * NWORK + w

    def load_start(k):
        buf, slot, b = k % 2, k % 4, blk_of(k)
        pltpu.async_copy(batch_hbm.at[pl.ds(b * BLK, BLK)], idxv.at[slot],
                         lsem[buf])
        pltpu.async_copy(x_hbm.at[pl.ds(b * BLK, BLK)], xblk.at[buf],
                         lsem[buf])

    def load_wait(k):
        buf, slot, b = k % 2, k % 4, blk_of(k)
        pltpu.make_async_copy(batch_hbm.at[pl.ds(b * BLK, BLK)],
                              idxv.at[slot], lsem[buf]).wait()
        pltpu.make_async_copy(x_hbm.at[pl.ds(b * BLK, BLK)],
                              xblk.at[buf], lsem[buf]).wait()

    def scatter_start(k):
        buf, slot = k % 2, k % 4
        pltpu.async_copy(sblk.at[buf], acc.at[idxv.at[slot]], ssem[buf],
                         add=True)

    def scatter_wait(k):
        buf, slot = k % 2, k % 4
        pltpu.make_async_copy(sblk.at[buf], acc.at[idxv.at[slot]],
                              ssem[buf]).wait()

    def compute(k):
        buf, slot = k % 2, k % 4

        def row4(i, carry):
            rows = [i * 4 + u for u in range(4)]
            xrs = [[xblk[buf, r, pl.ds(LANES * j, LANES)] for j in range(NJ)]
                   for r in rows]
            tots = []
            for xr in xrs:
                ps = [xr[j] * wr[j] for j in range(NJ)]
                while len(ps) > 1:
                    ps = [ps[i2] + ps[i2 + 1] for i2 in range(0, len(ps), 2)]
                tots.append(ps[0])
            for p in perms:
                tots = [t + _xlane(t, p) for t in tots]
            evs = [jnp.exp(t) for t in tots]
            for r, xr, ev in zip(rows, xrs, evs):
                for j in range(NJ):
                    sblk[buf, r, pl.ds(LANES * j, LANES)] = xr[j] * ev
                estage[r, pl.ds(0, LANES)] = ev
            return carry

        lax.fori_loop(0, BLK // 4, row4, 0)

        for q in range(NQ):
            eq = plsc.load_gather(estage, [lane + (LANES * q), zeroi])
            idxr = idxv[slot, pl.ds(LANES * q, LANES)]
            plsc.addupdate_scatter(gsum, [idxr], eq)

    conds = [blk_of(k) < NBLK for k in range(STEPS)]

    pl.when(conds[0])(lambda: load_start(0))
    for k in range(STEPS):
        if k + 1 < STEPS:
            pl.when(conds[k + 1])(lambda k=k: load_start(k + 1))
        if k >= 2:
            pl.when(conds[k - 2])(lambda k=k: scatter_wait(k - 2))

        def step(k=k):
            load_wait(k)
            compute(k)
            scatter_start(k)

        pl.when(conds[k])(step)

    for j in (STEPS - 2, STEPS - 1):
        pl.when(conds[j])(lambda j=j: scatter_wait(j))

    plsc.subcore_barrier()
    pltpu.sync_copy(acc.at[pl.ds(s * 32, 32)], out_hbm.at[c, pl.ds(s * 32, 32)])
    pltpu.sync_copy(gsum, gout_hbm.at[c, s])


def _finish_body(p_ref, g_ref, w_ref, b_ref, o_ref):
    a = p_ref[0] + p_ref[1]
    gs = jnp.sum(g_ref[...], axis=(0, 1))
    gsc = jnp.transpose(gs.reshape(1, NUM_G))
    nonempty = gsc > 0.0
    inv = jnp.where(nonempty, 1.0 / jnp.where(nonempty, gsc, 1.0), 0.0)
    pooled = a * inv
    out = jnp.dot(pooled, w_ref[...], preferred_element_type=jnp.float32)
    o_ref[...] = out + jnp.where(nonempty, b_ref[...], 0.0)


def kernel(x, batch, W_gate, b_gate, W_nn, b_nn):
    del b_gate
    batch32 = batch.astype(jnp.int32)
    wg = W_gate.reshape(IN_CH)

    mesh = plsc.VectorSubcoreMesh(core_axis_name="c", subcore_axis_name="s")
    sc_pool = pl.kernel(
        _sc_pool_body,
        mesh=mesh,
        compiler_params=pltpu.CompilerParams(
            needs_layout_passes=False, use_tc_tiling_on_sc=False),
        out_type=(
            jax.ShapeDtypeStruct((2, NUM_G, IN_CH), jnp.float32),
            jax.ShapeDtypeStruct((2, LANES, NUM_G), jnp.float32),
        ),
        scratch_types=[
            pltpu.VMEM((IN_CH,), jnp.float32),
            pltpu.VMEM((4, BLK), jnp.int32),
            pltpu.VMEM((2, BLK, IN_CH), jnp.float32),
            pltpu.VMEM((2, BLK, IN_CH), jnp.float32),
            pltpu.VMEM((BLK, LANES), jnp.float32),
            pltpu.VMEM((NUM_G,), jnp.float32),
            pltpu.VMEM_SHARED((NUM_G, IN_CH), jnp.float32),
            pltpu.SemaphoreType.DMA,
            pltpu.SemaphoreType.DMA,
            pltpu.SemaphoreType.DMA,
            pltpu.SemaphoreType.DMA,
        ],
    )
    partials, gparts = sc_pool(x, batch32, wg)

    out = pl.pallas_call(
        _finish_body,
        out_shape=jax.ShapeDtypeStruct((NUM_G, 2 * IN_CH), jnp.float32),
    )(partials, gparts, W_nn, b_nn.reshape(1, 2 * IN_CH))
    return out
